# baseline (speedup 1.0000x reference)
import jax
import jax.numpy as jnp
from jax import lax
from jax.experimental import pallas as pl
from jax.experimental.pallas import tpu as pltpu

G0, G1 = 64, 128
KN = G0 * G1
TN = 256


def _wprep_kernel(w_ref, wc_ref):
    d = w_ref.shape[0]
    wf = w_ref[...]
    wh = wf.astype(jnp.bfloat16)
    wl = (wf - wh.astype(jnp.float32)).astype(jnp.bfloat16)
    w2 = jnp.sum(wf * wf, axis=0, keepdims=True)
    w2h = w2.astype(jnp.bfloat16)
    w2l = (w2 - w2h.astype(jnp.float32)).astype(jnp.bfloat16)
    pad = jnp.zeros((wc_ref.shape[0] - 3 * d - 2, wf.shape[1]), jnp.bfloat16)
    wc_ref[...] = jnp.concatenate([wh, wh, wl, w2h, w2l, pad], axis=0)


def _som_kernel(x_ref, wc_ref, out_ref):
    x = x_ref[...]
    tn, d = x.shape
    x2 = jnp.sum(x * x, axis=1, keepdims=True)
    xs = -2.0 * x
    xh = xs.astype(jnp.bfloat16)
    xl = (xs - xh.astype(jnp.float32)).astype(jnp.bfloat16)
    ones = jnp.ones((tn, 1), jnp.bfloat16)
    zpad = jnp.zeros((tn, wc_ref.shape[0] - 3 * d - 2), jnp.bfloat16)
    xc = jnp.concatenate([xh, xl, xh, ones, ones, zpad], axis=1)
    dn = (((1,), (0,)), ((), ()))
    a = lax.dot_general(xc, wc_ref[...], dn,
                        preferred_element_type=jnp.float32)
    m = jnp.min(a, axis=1, keepdims=True)
    kiota = lax.broadcasted_iota(jnp.int32, (TN, KN), 1).astype(jnp.float32)
    sel = jnp.where(a == m, kiota, float(KN))
    idx = jnp.min(sel, axis=1, keepdims=True).astype(jnp.int32)
    p0 = idx // G1
    p1 = idx - p0 * G1
    iu = lax.broadcasted_iota(jnp.int32, (TN, G0), 1)
    iv = lax.broadcasted_iota(jnp.int32, (TN, G1), 1)
    du = (iu - p0).astype(jnp.float32)
    dv = (iv - p1).astype(jnp.float32)
    u = jnp.exp(-(du * du))
    v = jnp.exp(-(dv * dv))
    acc = a[:, 0:G1] * u[:, 0:1]
    for i in range(1, G0):
        acc = acc + a[:, i * G1:(i + 1) * G1] * u[:, i:i + 1]
    s = jnp.sum(u, axis=1, keepdims=True) * jnp.sum(v, axis=1, keepdims=True)
    out_ref[...] = x2 * s + jnp.sum(acc * v, axis=1, keepdims=True)


def kernel(x, w):
    n, d = x.shape
    dc = -(-(3 * d + 2) // 16) * 16
    wc = pl.pallas_call(
        _wprep_kernel,
        out_shape=jax.ShapeDtypeStruct((dc, KN), jnp.bfloat16),
    )(w)
    out = pl.pallas_call(
        _som_kernel,
        grid=(n // TN,),
        in_specs=[
            pl.BlockSpec((TN, d), lambda i: (i, 0)),
            pl.BlockSpec((dc, KN), lambda i: (0, 0)),
        ],
        out_specs=pl.BlockSpec((TN, 1), lambda i: (i, 0)),
        out_shape=jax.ShapeDtypeStruct((n, 1), jnp.float32),
    )(x, wc)
    return out[:, 0]


# speedup vs baseline: 1.2236x; 1.2236x over previous
import jax
import jax.numpy as jnp
from jax import lax
from jax.experimental import pallas as pl
from jax.experimental.pallas import tpu as pltpu

G0, G1 = 64, 128
KN = G0 * G1
TN = 256


def _som_kernel(x_ref, w_ref, out_ref, wh_ref, wl_ref, w2_ref):
    @pl.when(pl.program_id(0) == 0)
    def _():
        wf = w_ref[...]
        w2_ref[...] = jnp.sum(wf * wf, axis=0, keepdims=True)
        wh = wf.astype(jnp.bfloat16)
        wh_ref[...] = wh
        wl_ref[...] = (wf - wh.astype(jnp.float32)).astype(jnp.bfloat16)

    x = x_ref[...]
    x2 = jnp.sum(x * x, axis=1, keepdims=True)
    xs = -2.0 * x
    xh = xs.astype(jnp.bfloat16)
    xl = (xs - xh.astype(jnp.float32)).astype(jnp.bfloat16)
    dn = (((1,), (0,)), ((), ()))
    wh, wl = wh_ref[...], wl_ref[...]
    dot = (lax.dot_general(xh, wh, dn, preferred_element_type=jnp.float32)
           + lax.dot_general(xh, wl, dn, preferred_element_type=jnp.float32)
           + lax.dot_general(xl, wh, dn, preferred_element_type=jnp.float32))
    a = dot + w2_ref[...]
    m = jnp.min(a, axis=1, keepdims=True)
    kiota = lax.broadcasted_iota(jnp.int32, (TN, KN), 1).astype(jnp.float32)
    sel = jnp.where(a == m, kiota, float(KN))
    idx = jnp.min(sel, axis=1, keepdims=True).astype(jnp.int32)
    p0 = idx // G1
    p1 = idx - p0 * G1
    iu = lax.broadcasted_iota(jnp.int32, (TN, G0), 1)
    iv = lax.broadcasted_iota(jnp.int32, (TN, G1), 1)
    du = (iu - p0).astype(jnp.float32)
    dv = (iv - p1).astype(jnp.float32)
    u = jnp.exp(-(du * du))
    v = jnp.exp(-(dv * dv))
    acc = a[:, 0:G1] * u[:, 0:1]
    for i in range(1, G0):
        acc = acc + a[:, i * G1:(i + 1) * G1] * u[:, i:i + 1]
    s = jnp.sum(u, axis=1, keepdims=True) * jnp.sum(v, axis=1, keepdims=True)
    out_ref[...] = x2 * s + jnp.sum(acc * v, axis=1, keepdims=True)


def kernel(x, w):
    n, d = x.shape
    out = pl.pallas_call(
        _som_kernel,
        grid=(n // TN,),
        in_specs=[
            pl.BlockSpec((TN, d), lambda i: (i, 0)),
            pl.BlockSpec((d, KN), lambda i: (0, 0)),
        ],
        out_specs=pl.BlockSpec((TN, 1), lambda i: (i, 0)),
        out_shape=jax.ShapeDtypeStruct((n, 1), jnp.float32),
        scratch_shapes=[
            pltpu.VMEM((d, KN), jnp.bfloat16),
            pltpu.VMEM((d, KN), jnp.bfloat16),
            pltpu.VMEM((1, KN), jnp.float32),
        ],
    )(x, w)
    return out[:, 0]
